# Initial kernel scaffold; baseline (speedup 1.0000x reference)
import jax
import jax.numpy as jnp
from jax.experimental import pallas as pl


def kernel(x, temperature, qkv_w, dw_w, proj_w):
    raise NotImplementedError("write your pallas kernel here")


# speedup vs baseline: 120.6721x; 120.6721x over previous
import jax
import jax.numpy as jnp
from jax import lax
from jax.experimental import pallas as pl
from jax.experimental.pallas import tpu as pltpu

GROUP = 4
HEADS = 8
K_FRAC = 0.8
CH = 32
TSEQ = 2304
HW = 576
RT = 256
NTILES = TSEQ // RT
KK = int(TSEQ * K_FRAC)


def _attn_body(temp_ref, q_ref, k_ref, v_ref, o_ref):
    hh = pl.program_id(0)
    tt = pl.program_id(1)
    q = q_ref[0]
    k = k_ref[0]
    v = v_ref[0]
    tiny = jnp.float32(1e-12)
    qinv = 1.0 / jnp.maximum(jnp.sqrt(jnp.sum(q * q, axis=1, keepdims=True)), tiny)
    kinv = 1.0 / jnp.maximum(jnp.sqrt(jnp.sum(k * k, axis=1, keepdims=True)), tiny)
    kb = (k * kinv).astype(jnp.bfloat16)
    qb = (q_ref[0, :, pl.ds(tt * RT, RT)] * qinv).astype(jnp.bfloat16)
    s = lax.dot_general(qb, kb, (((0,), (0,)), ((), ())),
                        preferred_element_type=jnp.float32) * temp_ref[hh]

    minint = jnp.int32(-(2**31))
    mask31 = jnp.int32(0x7FFFFFFF)
    m = lax.bitcast_convert_type(s, jnp.int32)
    ikey = m ^ (lax.shift_right_arithmetic(m, 31) & mask31)

    def body(i, prefix):
        bit = lax.shift_left(jnp.int32(1), jnp.int32(31) - i)
        trial = prefix | bit
        cnt = jnp.sum((ikey >= (trial ^ minint)).astype(jnp.int32),
                      axis=1, keepdims=True)
        return jnp.where(cnt >= KK, trial, prefix)

    prefix = lax.fori_loop(0, 32, body, jnp.zeros((RT, 1), jnp.int32))
    thr = prefix ^ minint

    rowmax = jnp.max(s, axis=1, keepdims=True)
    p = jnp.where(ikey >= thr, jnp.exp(s - rowmax), jnp.float32(0.0))
    denom = jnp.sum(p, axis=1, keepdims=True)
    o_ref[0] = lax.dot_general(p, v, (((1,), (1,)), ((), ())),
                               precision=lax.Precision.HIGHEST) / denom


def _proj_body(w_ref, a_ref, o_ref):
    o_ref[...] = jnp.dot(w_ref[...], a_ref[...], preferred_element_type=jnp.float32,
                         precision=lax.Precision.HIGHEST)


def kernel(x, temperature, qkv_w, dw_w, proj_w):
    b, c, h, w = x.shape
    t = GROUP
    cg = c // t
    x5 = x.reshape(b, t, cg, h, w)

    qkv = jnp.einsum('oi,bidhw->bodhw', qkv_w[:, :, 0, 0, 0], x5)
    qkv = lax.conv_general_dilated(
        qkv, dw_w, window_strides=(1, 1, 1),
        padding=((0, 0), (1, 1), (1, 1)),
        dimension_numbers=('NCDHW', 'OIDHW', 'NCDHW'),
        feature_group_count=3 * t)

    qkvh = qkv.reshape(3, t, HEADS, CH, HW)
    qkvh = jnp.transpose(qkvh, (0, 2, 3, 4, 1)).reshape(3, HEADS, CH, TSEQ)
    qh, kh, vh = qkvh[0], qkvh[1], qkvh[2]
    temp = temperature.reshape(HEADS)

    out_attn = pl.pallas_call(
        _attn_body,
        grid=(HEADS, NTILES),
        in_specs=[
            pl.BlockSpec(memory_space=pltpu.SMEM),
            pl.BlockSpec((1, CH, TSEQ), lambda hh, tt: (hh, 0, 0)),
            pl.BlockSpec((1, CH, TSEQ), lambda hh, tt: (hh, 0, 0)),
            pl.BlockSpec((1, CH, TSEQ), lambda hh, tt: (hh, 0, 0)),
        ],
        out_specs=pl.BlockSpec((1, RT, CH), lambda hh, tt: (hh, tt, 0)),
        out_shape=jax.ShapeDtypeStruct((HEADS, TSEQ, CH), jnp.float32),
    )(temp, qh, kh, vh)

    a = jnp.transpose(out_attn.reshape(HEADS, HW, t, CH), (2, 0, 3, 1)).reshape(c, HW)
    y = pl.pallas_call(
        _proj_body,
        out_shape=jax.ShapeDtypeStruct((c, HW), jnp.float32),
    )(proj_w[:, :, 0, 0], a)
    return y.reshape(b, c, h, w)
